# baseline (speedup 1.0000x reference)
import jax
import jax.numpy as jnp
from jax import lax
from jax.experimental import pallas as pl
from jax.experimental.pallas import tpu as pltpu
from jax.experimental.pallas import tpu_sc as plsc

L_SEQ, B_SZ = 200, 1024
N_TOK = L_SEQ * B_SZ
NC, NS = 2, 16
NW = NC * NS
PER_W = N_TOK // NW
CHUNK = 128
NCH = PER_W // CHUNK
WIDTHS = (64, 64, 32, 32, 32, 32, 32)
COLS = (0, 64, 128, 160, 192, 224, 256)
OUT_D = 288
NT = 7


def _body(*refs):
    tables = refs[0:NT]
    idx_hbm = refs[NT:2 * NT]
    out = refs[2 * NT]
    idx_v = refs[2 * NT + 1:3 * NT + 1]
    rows0 = refs[3 * NT + 1:4 * NT + 1]
    rows1 = refs[4 * NT + 1:5 * NT + 1]
    rows = (rows0, rows1)
    sem_g = refs[5 * NT + 1:5 * NT + 3]
    sem_w = refs[5 * NT + 3:5 * NT + 5]

    wid = lax.axis_index("s") * NC + lax.axis_index("c")

    for t in range(NT):
        pltpu.sync_copy(idx_hbm[t].at[wid], idx_v[t])

    def fire_gathers(chunk, bufs, sem):
        for t in range(NT):
            pltpu.async_copy(tables[t].at[idx_v[t].at[chunk]], bufs[t], sem)

    def wait_gathers(bufs, sem):
        for t in range(NT):
            pltpu.make_async_copy(tables[t].at[idx_v[t].at[0]], bufs[t],
                                  sem).wait()

    def fire_write(chunk, bufs, sem):
        for t in range(NT):
            pltpu.async_copy(
                bufs[t], out.at[wid, chunk, slice(None),
                                pl.ds(COLS[t], WIDTHS[t])], sem)

    def wait_write(bufs, sem):
        for t in range(NT):
            pltpu.make_async_copy(
                bufs[t], out.at[wid, 0, slice(None),
                                pl.ds(COLS[t], WIDTHS[t])], sem).wait()

    fire_gathers(0, rows[0], sem_g[0])

    n_iter = NCH // 2

    def body(j, carry):
        a = 2 * j
        wait_gathers(rows[0], sem_g[0])
        fire_write(a, rows[0], sem_w[0])

        @pl.when(j > 0)
        def _():
            wait_write(rows[1], sem_w[1])

        fire_gathers(a + 1, rows[1], sem_g[1])
        wait_write(rows[0], sem_w[0])

        @pl.when(j < n_iter - 1)
        def _():
            fire_gathers(a + 2, rows[0], sem_g[0])

        wait_gathers(rows[1], sem_g[1])
        fire_write(a + 1, rows[1], sem_w[1])
        return carry

    lax.fori_loop(0, n_iter, body, 0)
    wait_write(rows[1], sem_w[1])


def kernel(attribute_key, attribute_word, attribute_word_local_fw_pos,
           attribute_word_local_bw_pos, attribute_kv_pos, attribute_kw_pos,
           attribute_word_tag, field_key_table, field_word_table,
           local_pos_fw_table, local_pos_bw_table, kv_pos_table,
           kw_pos_table, field_tag_table):
    tables = (field_word_table, field_key_table, local_pos_fw_table,
              local_pos_bw_table, kv_pos_table, kw_pos_table, field_tag_table)
    idx_arrays = (attribute_word, attribute_key, attribute_word_local_fw_pos,
                  attribute_word_local_bw_pos, attribute_kv_pos,
                  attribute_kw_pos, attribute_word_tag)
    idxs = [a.reshape(NW, NCH, CHUNK) for a in idx_arrays]

    mesh = plsc.VectorSubcoreMesh(core_axis_name="c", subcore_axis_name="s")
    scratch = (
        [pltpu.VMEM((NCH, CHUNK), jnp.int32) for _ in range(NT)]
        + [pltpu.VMEM((CHUNK, w), jnp.float32) for w in WIDTHS]
        + [pltpu.VMEM((CHUNK, w), jnp.float32) for w in WIDTHS]
        + [pltpu.SemaphoreType.DMA for _ in range(4)]
    )
    out = pl.kernel(
        _body,
        out_type=jax.ShapeDtypeStruct((NW, NCH, CHUNK, OUT_D), jnp.float32),
        mesh=mesh,
        scratch_types=scratch,
        compiler_params=pltpu.CompilerParams(use_tc_tiling_on_sc=False),
    )(*tables, *idxs)
    return out.reshape(L_SEQ, B_SZ, OUT_D)


# speedup vs baseline: 1.0008x; 1.0008x over previous
import jax
import jax.numpy as jnp
from jax import lax
from jax.experimental import pallas as pl
from jax.experimental.pallas import tpu as pltpu
from jax.experimental.pallas import tpu_sc as plsc

L_SEQ, B_SZ = 200, 1024
NC, NS = 2, 16
NW = NC * NS
CHUNK = 128
NBB = B_SZ // CHUNK
NCH = L_SEQ * NBB // NW
WIDTHS = (64, 64, 32, 32, 32, 32, 32)
COLS = (0, 64, 128, 160, 192, 224, 256)
OUT_D = 288
NT = 7


def _body(*refs):
    tables = refs[0:NT]
    idx_hbm = refs[NT:2 * NT]
    out = refs[2 * NT]
    idx_v = refs[2 * NT + 1:3 * NT + 1]
    rows0 = refs[3 * NT + 1:4 * NT + 1]
    rows1 = refs[4 * NT + 1:5 * NT + 1]
    rows = (rows0, rows1)
    sem_g = refs[5 * NT + 1:5 * NT + 3]
    sem_w = refs[5 * NT + 3:5 * NT + 5]

    wid = lax.axis_index("s") * NC + lax.axis_index("c")
    l0 = (wid // NBB) * NCH
    b0 = (wid % NBB) * CHUNK

    for t in range(NT):
        pltpu.sync_copy(idx_hbm[t].at[pl.ds(l0, NCH), pl.ds(b0, CHUNK)],
                        idx_v[t])

    def fire_gathers(chunk, bufs, sem):
        for t in range(NT):
            pltpu.async_copy(tables[t].at[idx_v[t].at[chunk]], bufs[t], sem)

    def wait_gathers(bufs, sem):
        for t in range(NT):
            pltpu.make_async_copy(tables[t].at[idx_v[t].at[0]], bufs[t],
                                  sem).wait()

    def fire_write(chunk, bufs, sem):
        for t in range(NT):
            pltpu.async_copy(
                bufs[t], out.at[l0 + chunk, pl.ds(b0, CHUNK),
                                pl.ds(COLS[t], WIDTHS[t])], sem)

    def wait_write(bufs, sem):
        for t in range(NT):
            pltpu.make_async_copy(
                bufs[t], out.at[l0, pl.ds(b0, CHUNK),
                                pl.ds(COLS[t], WIDTHS[t])], sem).wait()

    fire_gathers(0, rows[0], sem_g[0])

    n_iter = NCH // 2

    def body(j, carry):
        a = 2 * j
        wait_gathers(rows[0], sem_g[0])
        fire_write(a, rows[0], sem_w[0])

        @pl.when(j > 0)
        def _():
            wait_write(rows[1], sem_w[1])

        fire_gathers(a + 1, rows[1], sem_g[1])
        wait_write(rows[0], sem_w[0])

        @pl.when(j < n_iter - 1)
        def _():
            fire_gathers(a + 2, rows[0], sem_g[0])

        wait_gathers(rows[1], sem_g[1])
        fire_write(a + 1, rows[1], sem_w[1])
        return carry

    lax.fori_loop(0, n_iter, body, 0)
    wait_write(rows[1], sem_w[1])


def kernel(attribute_key, attribute_word, attribute_word_local_fw_pos,
           attribute_word_local_bw_pos, attribute_kv_pos, attribute_kw_pos,
           attribute_word_tag, field_key_table, field_word_table,
           local_pos_fw_table, local_pos_bw_table, kv_pos_table,
           kw_pos_table, field_tag_table):
    tables = (field_word_table, field_key_table, local_pos_fw_table,
              local_pos_bw_table, kv_pos_table, kw_pos_table, field_tag_table)
    idxs = (attribute_word, attribute_key, attribute_word_local_fw_pos,
            attribute_word_local_bw_pos, attribute_kv_pos,
            attribute_kw_pos, attribute_word_tag)

    mesh = plsc.VectorSubcoreMesh(core_axis_name="c", subcore_axis_name="s")
    scratch = (
        [pltpu.VMEM((NCH, CHUNK), jnp.int32) for _ in range(NT)]
        + [pltpu.VMEM((CHUNK, w), jnp.float32) for w in WIDTHS]
        + [pltpu.VMEM((CHUNK, w), jnp.float32) for w in WIDTHS]
        + [pltpu.SemaphoreType.DMA for _ in range(4)]
    )
    return pl.kernel(
        _body,
        out_type=jax.ShapeDtypeStruct((L_SEQ, B_SZ, OUT_D), jnp.float32),
        mesh=mesh,
        scratch_types=scratch,
        compiler_params=pltpu.CompilerParams(use_tc_tiling_on_sc=False),
    )(*tables, *idxs)
